# baseline (speedup 1.0000x reference)
import functools

import jax
import jax.numpy as jnp
from jax import lax
from jax.experimental import pallas as pl
from jax.experimental.pallas import tpu as pltpu
from jax.experimental.pallas import tpu_sc as plsc

V = 256
B = 16384
SEQ_LEN = 50
L = 16
NW = 32
BPW = B // NW
C = 32
NCHUNK = BPW // C
NSLOT = 4
A_TERT, A_BIN, A_UNARY = 0.5, 0.3, 0.2


def _tri_gram_body(text_hbm, unary_hbm, binary_hbm, tert_hbm,
                   out_hbm, prev_v, last_v, idxt_v, idxb_v, ubuf,
                   tbuf0, tbuf1, tbuf2, tbuf3, bbuf0, bbuf1, bbuf2, bbuf3,
                   sem_t0, sem_t1, sem_t2, sem_t3,
                   sem_b0, sem_b1, sem_b2, sem_b3,
                   sem_o0, sem_o1, sem_o2, sem_o3):
    wid = lax.axis_index("s") * 2 + lax.axis_index("c")
    base = wid * BPW
    tbuf = (tbuf0, tbuf1, tbuf2, tbuf3)
    bbuf = (bbuf0, bbuf1, bbuf2, bbuf3)
    sem_t = (sem_t0, sem_t1, sem_t2, sem_t3)
    sem_b = (sem_b0, sem_b1, sem_b2, sem_b3)
    sem_o = (sem_o0, sem_o1, sem_o2, sem_o3)

    pltpu.sync_copy(text_hbm.at[SEQ_LEN - 2, pl.ds(base, BPW)], prev_v)
    pltpu.sync_copy(text_hbm.at[SEQ_LEN - 1, pl.ds(base, BPW)], last_v)
    pltpu.sync_copy(unary_hbm, ubuf)

    def fire(c, slot):
        return (
            pltpu.async_copy(tert_hbm.at[idxt_v.at[c]], tbuf[slot],
                             sem_t[slot]),
            pltpu.async_copy(binary_hbm.at[idxb_v.at[c]], bbuf[slot],
                             sem_b[slot]),
        )

    cp_g = [None] * NSLOT
    cp_o = [None] * NSLOT

    for c in range(NCHUNK):
        for i in range(C // L):
            off = c * C + i * L
            p = prev_v[pl.ds(off, L)]
            t = last_v[pl.ds(off, L)]
            idxt_v[c, pl.ds(i * L, L)] = p * V + t
            idxb_v[c, pl.ds(i * L, L)] = t
        if c < 2:
            cp_g[c] = fire(c, c)

    u = [ubuf[pl.ds(j * L, L)] * A_UNARY for j in range(V // L)]

    for c in range(NCHUNK):
        p = c % NSLOT
        cp_g[p][0].wait()
        cp_g[p][1].wait()
        tb, bb = tbuf[p], bbuf[p]

        def row_body(r, carry):
            for j in range(V // L):
                t = tb[r, pl.ds(j * L, L)]
                b = bb[r, pl.ds(j * L, L)]
                tb[r, pl.ds(j * L, L)] = t * A_TERT + b * A_BIN + u[j]
            return carry

        lax.fori_loop(0, C, row_body, 0)
        cp_o[p] = pltpu.async_copy(tb, out_hbm.at[pl.ds(base + c * C, C)],
                                   sem_o[p])
        nxt = c + 2
        if nxt < NCHUNK:
            s = nxt % NSLOT
            if cp_o[s] is not None:
                cp_o[s].wait()
                cp_o[s] = None
            cp_g[s] = fire(nxt, s)

    for q in range(NSLOT):
        if cp_o[q] is not None:
            cp_o[q].wait()


@jax.jit
def _tri_gram(text, unary, binary, tert2):
    mesh = plsc.VectorSubcoreMesh(core_axis_name="c", subcore_axis_name="s")
    return pl.kernel(
        _tri_gram_body,
        mesh=mesh,
        out_type=jax.ShapeDtypeStruct((B, V), jnp.float32),
        scratch_types=[
            pltpu.VMEM((BPW,), jnp.int32),
            pltpu.VMEM((BPW,), jnp.int32),
            pltpu.VMEM((NCHUNK, C), jnp.int32),
            pltpu.VMEM((NCHUNK, C), jnp.int32),
            pltpu.VMEM((V,), jnp.float32),
        ] + [pltpu.VMEM((C, V), jnp.float32)] * (2 * NSLOT)
          + [pltpu.SemaphoreType.DMA] * (3 * NSLOT),
    )(text, unary, binary, tert2)


def kernel(text, unary_counts, binary_counts, tert_counts):
    tert2 = tert_counts.reshape(V * V, V)
    return _tri_gram(text, unary_counts, binary_counts, tert2)


# speedup vs baseline: 1.0341x; 1.0341x over previous
import functools

import jax
import jax.numpy as jnp
from jax import lax
from jax.experimental import pallas as pl
from jax.experimental.pallas import tpu as pltpu
from jax.experimental.pallas import tpu_sc as plsc

V = 256
B = 16384
SEQ_LEN = 50
L = 16
NW = 32
BPW = B // NW
C = 64
NCHUNK = BPW // C
NSLOT = 3
A_TERT, A_BIN, A_UNARY = 0.5, 0.3, 0.2


def _tri_gram_body(text_hbm, unary_hbm, binary_hbm, tert_hbm,
                   out_hbm, prev_v, last_v, idxt_v, idxb_v, ubuf,
                   tbuf0, tbuf1, tbuf2, bbuf0, bbuf1, bbuf2,
                   sem_t0, sem_t1, sem_t2, sem_b0, sem_b1, sem_b2,
                   sem_o0, sem_o1, sem_o2):
    wid = lax.axis_index("s") * 2 + lax.axis_index("c")
    base = wid * BPW
    tbuf = (tbuf0, tbuf1, tbuf2)
    bbuf = (bbuf0, bbuf1, bbuf2)
    sem_t = (sem_t0, sem_t1, sem_t2)
    sem_b = (sem_b0, sem_b1, sem_b2)
    sem_o = (sem_o0, sem_o1, sem_o2)

    pltpu.sync_copy(text_hbm.at[SEQ_LEN - 2, pl.ds(base, BPW)], prev_v)
    pltpu.sync_copy(text_hbm.at[SEQ_LEN - 1, pl.ds(base, BPW)], last_v)
    pltpu.sync_copy(unary_hbm, ubuf)

    def fire(c, slot):
        return (
            pltpu.async_copy(tert_hbm.at[idxt_v.at[c]], tbuf[slot],
                             sem_t[slot]),
            pltpu.async_copy(binary_hbm.at[idxb_v.at[c]], bbuf[slot],
                             sem_b[slot]),
        )

    cp_g = [None, None, None]
    cp_o = [None, None, None]

    for c in range(NCHUNK):
        for i in range(C // L):
            off = c * C + i * L
            p = prev_v[pl.ds(off, L)]
            t = last_v[pl.ds(off, L)]
            idxt_v[c, pl.ds(i * L, L)] = p * V + t
            idxb_v[c, pl.ds(i * L, L)] = t
        if c < 2:
            cp_g[c] = fire(c, c)

    u = [ubuf[pl.ds(j * L, L)] * A_UNARY for j in range(V // L)]

    for c in range(NCHUNK):
        p = c % NSLOT
        cp_g[p][0].wait()
        cp_g[p][1].wait()
        tb, bb = tbuf[p], bbuf[p]

        def row_body(r, carry):
            for j in range(V // L):
                t = tb[r, pl.ds(j * L, L)]
                b = bb[r, pl.ds(j * L, L)]
                tb[r, pl.ds(j * L, L)] = t * A_TERT + b * A_BIN + u[j]
            return carry

        lax.fori_loop(0, C, row_body, 0)
        cp_o[p] = pltpu.async_copy(tb, out_hbm.at[pl.ds(base + c * C, C)],
                                   sem_o[p])
        nxt = c + 2
        if nxt < NCHUNK:
            s = nxt % NSLOT
            if cp_o[s] is not None:
                cp_o[s].wait()
                cp_o[s] = None
            cp_g[s] = fire(nxt, s)

    for q in range(NSLOT):
        if cp_o[q] is not None:
            cp_o[q].wait()


@jax.jit
def _tri_gram(text, unary, binary, tert2):
    mesh = plsc.VectorSubcoreMesh(core_axis_name="c", subcore_axis_name="s")
    return pl.kernel(
        _tri_gram_body,
        mesh=mesh,
        out_type=jax.ShapeDtypeStruct((B, V), jnp.float32),
        scratch_types=[
            pltpu.VMEM((BPW,), jnp.int32),
            pltpu.VMEM((BPW,), jnp.int32),
            pltpu.VMEM((NCHUNK, C), jnp.int32),
            pltpu.VMEM((NCHUNK, C), jnp.int32),
            pltpu.VMEM((V,), jnp.float32),
            pltpu.VMEM((C, V), jnp.float32),
            pltpu.VMEM((C, V), jnp.float32),
            pltpu.VMEM((C, V), jnp.float32),
            pltpu.VMEM((C, V), jnp.float32),
            pltpu.VMEM((C, V), jnp.float32),
            pltpu.VMEM((C, V), jnp.float32),
            pltpu.SemaphoreType.DMA,
            pltpu.SemaphoreType.DMA,
            pltpu.SemaphoreType.DMA,
            pltpu.SemaphoreType.DMA,
            pltpu.SemaphoreType.DMA,
            pltpu.SemaphoreType.DMA,
            pltpu.SemaphoreType.DMA,
            pltpu.SemaphoreType.DMA,
            pltpu.SemaphoreType.DMA,
        ],
    )(text, unary, binary, tert2)


def kernel(text, unary_counts, binary_counts, tert_counts):
    tert2 = tert_counts.reshape(V * V, V)
    return _tri_gram(text, unary_counts, binary_counts, tert2)


# speedup vs baseline: 1.0511x; 1.0165x over previous
import functools

import jax
import jax.numpy as jnp
from jax import lax
from jax.experimental import pallas as pl
from jax.experimental.pallas import tpu as pltpu
from jax.experimental.pallas import tpu_sc as plsc

V = 256
B = 16384
SEQ_LEN = 50
L = 16
NW = 32
BPW = B // NW
C = 64
NCHUNK = BPW // C
NSLOT = 3
A_TERT, A_BIN, A_UNARY = 0.5, 0.3, 0.2


def _tri_gram_body(text_hbm, unary_hbm, binary_hbm, tert_hbm,
                   out_hbm, prev_v, last_v, idxt_v, ubuf,
                   tbuf0, tbuf1, tbuf2, bbuf0, bbuf1, bbuf2,
                   sem_t0, sem_t1, sem_t2, sem_b0, sem_b1, sem_b2,
                   sem_o0, sem_o1, sem_o2):
    wid = lax.axis_index("s") * 2 + lax.axis_index("c")
    base = wid * BPW
    tbuf = (tbuf0, tbuf1, tbuf2)
    bbuf = (bbuf0, bbuf1, bbuf2)
    sem_t = (sem_t0, sem_t1, sem_t2)
    sem_b = (sem_b0, sem_b1, sem_b2)
    sem_o = (sem_o0, sem_o1, sem_o2)

    cp_p = pltpu.async_copy(text_hbm.at[SEQ_LEN - 2, pl.ds(base, BPW)],
                            prev_v, sem_o0)
    cp_l = pltpu.async_copy(text_hbm.at[SEQ_LEN - 1, pl.ds(base, BPW)],
                            last_v, sem_o1)
    cp_u = pltpu.async_copy(unary_hbm, ubuf, sem_o2)
    cp_p.wait()
    cp_l.wait()

    def fire(c, slot):
        return (
            pltpu.async_copy(tert_hbm.at[idxt_v.at[c]], tbuf[slot],
                             sem_t[slot]),
            pltpu.async_copy(binary_hbm.at[last_v.at[pl.ds(c * C, C)]],
                             bbuf[slot], sem_b[slot]),
        )

    cp_g = [None, None, None]
    cp_o = [None, None, None]

    for c in range(NCHUNK):
        for i in range(C // L):
            off = c * C + i * L
            p = prev_v[pl.ds(off, L)]
            t = last_v[pl.ds(off, L)]
            idxt_v[c, pl.ds(i * L, L)] = p * V + t
        if c < 2:
            cp_g[c] = fire(c, c)

    cp_u.wait()
    u = [ubuf[pl.ds(j * L, L)] * A_UNARY for j in range(V // L)]

    for c in range(NCHUNK):
        p = c % NSLOT
        cp_g[p][0].wait()
        cp_g[p][1].wait()
        tb, bb = tbuf[p], bbuf[p]

        def row_body(r, carry):
            for j in range(V // L):
                t = tb[r, pl.ds(j * L, L)]
                b = bb[r, pl.ds(j * L, L)]
                tb[r, pl.ds(j * L, L)] = t * A_TERT + b * A_BIN + u[j]
            return carry

        lax.fori_loop(0, C, row_body, 0)
        cp_o[p] = pltpu.async_copy(tb, out_hbm.at[pl.ds(base + c * C, C)],
                                   sem_o[p])
        nxt = c + 2
        if nxt < NCHUNK:
            s = nxt % NSLOT
            if cp_o[s] is not None:
                cp_o[s].wait()
                cp_o[s] = None
            cp_g[s] = fire(nxt, s)

    for q in range(NSLOT):
        if cp_o[q] is not None:
            cp_o[q].wait()


@jax.jit
def _tri_gram(text, unary, binary, tert2):
    mesh = plsc.VectorSubcoreMesh(core_axis_name="c", subcore_axis_name="s")
    return pl.kernel(
        _tri_gram_body,
        mesh=mesh,
        out_type=jax.ShapeDtypeStruct((B, V), jnp.float32),
        scratch_types=[
            pltpu.VMEM((BPW,), jnp.int32),
            pltpu.VMEM((BPW,), jnp.int32),
            pltpu.VMEM((NCHUNK, C), jnp.int32),
            pltpu.VMEM((V,), jnp.float32),
            pltpu.VMEM((C, V), jnp.float32),
            pltpu.VMEM((C, V), jnp.float32),
            pltpu.VMEM((C, V), jnp.float32),
            pltpu.VMEM((C, V), jnp.float32),
            pltpu.VMEM((C, V), jnp.float32),
            pltpu.VMEM((C, V), jnp.float32),
            pltpu.SemaphoreType.DMA,
            pltpu.SemaphoreType.DMA,
            pltpu.SemaphoreType.DMA,
            pltpu.SemaphoreType.DMA,
            pltpu.SemaphoreType.DMA,
            pltpu.SemaphoreType.DMA,
            pltpu.SemaphoreType.DMA,
            pltpu.SemaphoreType.DMA,
            pltpu.SemaphoreType.DMA,
        ],
    )(text, unary, binary, tert2)


def kernel(text, unary_counts, binary_counts, tert_counts):
    tert2 = tert_counts.reshape(V * V, V)
    return _tri_gram(text, unary_counts, binary_counts, tert2)
